# baseline (speedup 1.0000x reference)
import functools
import math

import jax
import jax.numpy as jnp
from jax import lax
from jax.experimental import pallas as pl
from jax.experimental.pallas import tpu as pltpu

THETA_BINS = 8
PHI_BINS = 8
BLOCK = 15
PAD = BLOCK // 2
MAX_PHI = math.pi
EPS = 2.220446049250313e-16
N = 64
D = 78
NB = THETA_BINS * PHI_BINS
STEPS = D + PAD

_ATAN_COEFS = (1.0, -0.33333293, 0.19998533, -0.14264892, 0.109583646,
               -0.08427638, 0.058457974, -0.031750698, 0.011257721,
               -0.0018775827)
_HALF_PI = math.pi / 2


def _atan2(y, x):
    ax = jnp.abs(x)
    ay = jnp.abs(y)
    hi = jnp.maximum(ax, ay)
    lo = jnp.minimum(ax, ay)
    a = lo / jnp.where(hi == 0, 1.0, hi)
    s = a * a
    p = jnp.float32(_ATAN_COEFS[-1])
    for c in _ATAN_COEFS[-2::-1]:
        p = p * s + jnp.float32(c)
    t = a * p
    t = jnp.where(ay > ax, _HALF_PI - t, t)
    t = jnp.where(x < 0, math.pi - t, t)
    return jnp.where(y < 0, -t, t)


def _cnt(i):
    return (jnp.minimum(i, PAD) + jnp.minimum(D - 1 - i, PAD) + 1
            ).astype(jnp.float32)


def _hw_plane(xpad_ref, i):
    a = xpad_ref[pl.ds(i, 3), :, :]
    g0 = a[2, 1:79, 1:79] - a[0, 1:79, 1:79]
    g1 = a[1, 2:80, 1:79] - a[1, 0:78, 1:79]
    g2 = a[1, 1:79, 2:80] - a[1, 1:79, 0:78]

    mag2 = g0 * g0 + g1 * g1 + g2 * g2
    safe = mag2 > 0
    mag = jnp.where(safe, jnp.sqrt(jnp.where(safe, mag2, 1.0)), 0.0)
    ty = jnp.where(safe, g1, 0.0)
    tx = jnp.where(safe, g2, 1.0)
    theta = jnp.where(safe, _atan2(ty, tx), 0.0)
    ratio = jnp.clip(g0 / (mag + EPS), -1.0 + 1e-6, 1.0 - 1e-6)
    phi = _atan2(jnp.sqrt((1.0 - ratio) * (1.0 + ratio)), ratio)

    theta_raw = theta * (PHI_BINS / MAX_PHI)
    phi_raw = phi * (PHI_BINS / MAX_PHI)
    tf = theta_raw - jnp.where(theta_raw >= 0, jnp.floor(theta_raw),
                               jnp.ceil(theta_raw))
    pf = phi_raw - jnp.where(phi_raw >= 0, jnp.floor(phi_raw),
                             jnp.ceil(phi_raw))
    t0 = jnp.floor(theta_raw).astype(jnp.int32) & (THETA_BINS - 1)
    t1 = jnp.ceil(theta_raw).astype(jnp.int32) & (THETA_BINS - 1)
    p0 = jnp.floor(phi_raw).astype(jnp.int32) & (PHI_BINS - 1)
    p1 = jnp.ceil(phi_raw).astype(jnp.int32) & (PHI_BINS - 1)
    f0 = jnp.abs(tf)
    f1 = jnp.abs(1.0 - tf)
    f2 = jnp.abs(pf)
    f3 = jnp.abs(1.0 - pf)

    it = lax.broadcasted_iota(jnp.int32, (THETA_BINS, D, D), 0)
    T = (jnp.where(it == t0[None], f0[None], 0.0)
         + jnp.where(it == t1[None], f1[None], 0.0)) * mag[None]
    P = (jnp.where(it == p0[None], f2[None], 0.0)
         + jnp.where(it == p1[None], f3[None], 0.0))
    zt = jnp.zeros((THETA_BINS, 8, D), jnp.float32)
    zb = jnp.zeros((THETA_BINS, 10, D), jnp.float32)
    Tp = jnp.concatenate([zt, T, zb], axis=1)
    Pp = jnp.concatenate([zt, P, zb], axis=1)
    hist = (Tp[:, None] * Pp[None, :]).reshape(NB, 96, D)

    ir = lax.broadcasted_iota(jnp.int32, (D, D), 0)
    ic = lax.broadcasted_iota(jnp.int32, (D, D), 1)
    nw = (jnp.abs(ir - ic) <= PAD).astype(jnp.bfloat16)
    yw = jax.lax.dot(hist.reshape(NB * 96, D).astype(jnp.bfloat16), nw,
                     preferred_element_type=jnp.float32).reshape(NB, 96, D)
    acc = yw[:, 1:1 + D, :]
    for k in range(2, BLOCK + 1):
        acc = acc + yw[:, k:k + D, :]
    return acc


def _fused_body(xpad_ref, o_ref, ring_ref, s_ref):
    i = pl.program_id(0)

    @pl.when(i == 0)
    def _init():
        ring_ref[...] = jnp.zeros((BLOCK, NB, D, D), jnp.float32)
        s_ref[...] = jnp.zeros((NB, D, D), jnp.float32)

    pln = _hw_plane(xpad_ref, i)
    slot = lax.rem(i, BLOCK)
    old = ring_ref[slot]
    s_new = s_ref[...] + pln - old
    s_ref[...] = s_new
    ring_ref[slot] = pln

    @pl.when(i >= PAD)
    def _emit():
        z = i - PAD
        ih = lax.broadcasted_iota(jnp.int32, (D, D), 0)
        iw = lax.broadcasted_iota(jnp.int32, (D, D), 1)
        inv2 = (1.0 / _cnt(z)) / (_cnt(ih) * _cnt(iw))
        o_ref[:, 0] = s_new * inv2[None]


@functools.partial(jax.jit, static_argnames=("interpret",))
def _hog(x, weight, interpret=False):
    del weight
    xr = x.astype(jnp.bfloat16).astype(jnp.float32)
    xpad = jnp.pad(xr, ((8, 15), (8, 8), (8, 8)))
    out = pl.pallas_call(
        _fused_body,
        grid=(STEPS,),
        in_specs=[pl.BlockSpec((N + 23, N + 16, N + 16),
                               lambda i: (0, 0, 0))],
        out_specs=pl.BlockSpec(
            (NB, 1, D, D),
            lambda i: (0, jnp.where(i < PAD, 0, i - PAD), 0, 0)),
        out_shape=jax.ShapeDtypeStruct((NB, D, D, D), jnp.float32),
        scratch_shapes=[
            pltpu.VMEM((BLOCK, NB, D, D), jnp.float32),
            pltpu.VMEM((NB, D, D), jnp.float32),
        ],
        interpret=interpret,
    )(xpad)
    return out


def kernel(x, weight):
    return _hog(x, weight)


# speedup vs baseline: 1.1916x; 1.1916x over previous
import functools
import math

import jax
import jax.numpy as jnp
from jax import lax
from jax.experimental import pallas as pl
from jax.experimental.pallas import tpu as pltpu

THETA_BINS = 8
PHI_BINS = 8
BLOCK = 15
PAD = BLOCK // 2
MAX_PHI = math.pi
EPS = 2.220446049250313e-16
N = 64
D = 78
NB = THETA_BINS * PHI_BINS
STEPS = D + PAD

_ATAN_COEFS = (1.0, -0.33333293, 0.19998533, -0.14264892, 0.109583646,
               -0.08427638, 0.058457974, -0.031750698, 0.011257721,
               -0.0018775827)
_HALF_PI = math.pi / 2


def _atan2(y, x):
    ax = jnp.abs(x)
    ay = jnp.abs(y)
    hi = jnp.maximum(ax, ay)
    lo = jnp.minimum(ax, ay)
    a = lo / jnp.where(hi == 0, 1.0, hi)
    s = a * a
    p = jnp.float32(_ATAN_COEFS[-1])
    for c in _ATAN_COEFS[-2::-1]:
        p = p * s + jnp.float32(c)
    t = a * p
    t = jnp.where(ay > ax, _HALF_PI - t, t)
    t = jnp.where(x < 0, math.pi - t, t)
    return jnp.where(y < 0, -t, t)


def _cnt(i):
    return (jnp.minimum(i, PAD) + jnp.minimum(D - 1 - i, PAD) + 1
            ).astype(jnp.float32)


def _hw_plane(xpad_ref, i):
    a = xpad_ref[pl.ds(i, 3), :, :]
    g0 = a[2, 1:79, 1:79] - a[0, 1:79, 1:79]
    g1 = a[1, 2:80, 1:79] - a[1, 0:78, 1:79]
    g2 = a[1, 1:79, 2:80] - a[1, 1:79, 0:78]

    mag2 = g0 * g0 + g1 * g1 + g2 * g2
    safe = mag2 > 0
    mag = jnp.where(safe, jnp.sqrt(jnp.where(safe, mag2, 1.0)), 0.0)
    ty = jnp.where(safe, g1, 0.0)
    tx = jnp.where(safe, g2, 1.0)
    theta = jnp.where(safe, _atan2(ty, tx), 0.0)
    ratio = jnp.clip(g0 / (mag + EPS), -1.0 + 1e-6, 1.0 - 1e-6)
    phi = _atan2(jnp.sqrt((1.0 - ratio) * (1.0 + ratio)), ratio)

    theta_raw = theta * (PHI_BINS / MAX_PHI)
    phi_raw = phi * (PHI_BINS / MAX_PHI)
    tf = theta_raw - jnp.where(theta_raw >= 0, jnp.floor(theta_raw),
                               jnp.ceil(theta_raw))
    pf = phi_raw - jnp.where(phi_raw >= 0, jnp.floor(phi_raw),
                             jnp.ceil(phi_raw))
    t0 = jnp.floor(theta_raw).astype(jnp.int32) & (THETA_BINS - 1)
    t1 = jnp.ceil(theta_raw).astype(jnp.int32) & (THETA_BINS - 1)
    p0 = jnp.floor(phi_raw).astype(jnp.int32) & (PHI_BINS - 1)
    p1 = jnp.ceil(phi_raw).astype(jnp.int32) & (PHI_BINS - 1)
    f0 = jnp.abs(tf)
    f1 = jnp.abs(1.0 - tf)
    f2 = jnp.abs(pf)
    f3 = jnp.abs(1.0 - pf)

    it = lax.broadcasted_iota(jnp.int32, (THETA_BINS, D, D), 0)
    T = (jnp.where(it == t0[None], f0[None], 0.0)
         + jnp.where(it == t1[None], f1[None], 0.0)) * mag[None]
    P = (jnp.where(it == p0[None], f2[None], 0.0)
         + jnp.where(it == p1[None], f3[None], 0.0))
    zt = jnp.zeros((THETA_BINS, 8, D), jnp.float32)
    zb = jnp.zeros((THETA_BINS, 10, D), jnp.float32)
    Tp = jnp.concatenate([zt, T, zb], axis=1)
    Pp = jnp.concatenate([zt, P, zb], axis=1)
    hist = (Tp[:, None] * Pp[None, :]).reshape(NB, 96, D)

    ir = lax.broadcasted_iota(jnp.int32, (D, D), 0)
    ic = lax.broadcasted_iota(jnp.int32, (D, D), 1)
    nw = (jnp.abs(ir - ic) <= PAD).astype(jnp.float32)
    yw = jax.lax.dot(hist.reshape(NB * 96, D), nw).reshape(NB, 96, D)
    g = yw[:, 1:1 + 88, :]
    for k in range(2, 6):
        g = g + yw[:, k:k + 88, :]
    return g[:, 0:D, :] + g[:, 5:5 + D, :] + g[:, 10:10 + D, :]


def _fused_body(xpad_ref, o_ref, ring_ref, s_ref):
    i = pl.program_id(0)

    @pl.when(i == 0)
    def _init():
        ring_ref[...] = jnp.zeros((BLOCK, NB, D, D), jnp.float32)
        s_ref[...] = jnp.zeros((NB, D, D), jnp.float32)

    pln = _hw_plane(xpad_ref, i)
    slot = lax.rem(i, BLOCK)
    old = ring_ref[slot]
    s_new = s_ref[...] + pln - old
    s_ref[...] = s_new
    ring_ref[slot] = pln

    @pl.when(i >= PAD)
    def _emit():
        z = i - PAD
        ih = lax.broadcasted_iota(jnp.int32, (D, D), 0)
        iw = lax.broadcasted_iota(jnp.int32, (D, D), 1)
        inv2 = (1.0 / _cnt(z)) / (_cnt(ih) * _cnt(iw))
        o_ref[:, 0] = s_new * inv2[None]


@functools.partial(jax.jit, static_argnames=("interpret",))
def _hog(x, weight, interpret=False):
    del weight
    xr = x.astype(jnp.bfloat16).astype(jnp.float32)
    xpad = jnp.pad(xr, ((8, 15), (8, 8), (8, 8)))
    out = pl.pallas_call(
        _fused_body,
        grid=(STEPS,),
        in_specs=[pl.BlockSpec((N + 23, N + 16, N + 16),
                               lambda i: (0, 0, 0))],
        out_specs=pl.BlockSpec(
            (NB, 1, D, D),
            lambda i: (0, jnp.where(i < PAD, 0, i - PAD), 0, 0)),
        out_shape=jax.ShapeDtypeStruct((NB, D, D, D), jnp.float32),
        scratch_shapes=[
            pltpu.VMEM((BLOCK, NB, D, D), jnp.float32),
            pltpu.VMEM((NB, D, D), jnp.float32),
        ],
        interpret=interpret,
    )(xpad)
    return out


def kernel(x, weight):
    return _hog(x, weight)


# speedup vs baseline: 1.7275x; 1.4498x over previous
import functools
import math

import jax
import jax.numpy as jnp
from jax import lax
from jax.experimental import pallas as pl
from jax.experimental.pallas import tpu as pltpu

THETA_BINS = 8
PHI_BINS = 8
BLOCK = 15
PAD = BLOCK // 2
MAX_PHI = math.pi
EPS = 2.220446049250313e-16
N = 64
D = 78
NB = THETA_BINS * PHI_BINS
STEPS = D + PAD

_ATAN_COEFS = (1.0, -0.33333293, 0.19998533, -0.14264892, 0.109583646,
               -0.08427638, 0.058457974, -0.031750698, 0.011257721,
               -0.0018775827)
_HALF_PI = math.pi / 2


def _atan2(y, x):
    ax = jnp.abs(x)
    ay = jnp.abs(y)
    hi = jnp.maximum(ax, ay)
    lo = jnp.minimum(ax, ay)
    a = lo / jnp.where(hi == 0, 1.0, hi)
    s = a * a
    p = jnp.float32(_ATAN_COEFS[-1])
    for c in _ATAN_COEFS[-2::-1]:
        p = p * s + jnp.float32(c)
    t = a * p
    t = jnp.where(ay > ax, _HALF_PI - t, t)
    t = jnp.where(x < 0, math.pi - t, t)
    return jnp.where(y < 0, -t, t)


def _cnt(i):
    return (jnp.minimum(i, PAD) + jnp.minimum(D - 1 - i, PAD) + 1
            ).astype(jnp.float32)


def _hw_plane(xpad_ref, i):
    a = xpad_ref[pl.ds(i, 3), :, :]
    g0 = a[2, 1:79, 1:79] - a[0, 1:79, 1:79]
    g1 = a[1, 2:80, 1:79] - a[1, 0:78, 1:79]
    g2 = a[1, 1:79, 2:80] - a[1, 1:79, 0:78]

    mag2 = g0 * g0 + g1 * g1 + g2 * g2
    safe = mag2 > 0
    mag = jnp.where(safe, jnp.sqrt(jnp.where(safe, mag2, 1.0)), 0.0)
    ty = jnp.where(safe, g1, 0.0)
    tx = jnp.where(safe, g2, 1.0)
    theta = jnp.where(safe, _atan2(ty, tx), 0.0)
    ratio = jnp.clip(g0 / (mag + EPS), -1.0 + 1e-6, 1.0 - 1e-6)
    phi = _atan2(jnp.sqrt((1.0 - ratio) * (1.0 + ratio)), ratio)

    theta_raw = theta * (PHI_BINS / MAX_PHI)
    phi_raw = phi * (PHI_BINS / MAX_PHI)
    tf = theta_raw - jnp.where(theta_raw >= 0, jnp.floor(theta_raw),
                               jnp.ceil(theta_raw))
    pf = phi_raw - jnp.where(phi_raw >= 0, jnp.floor(phi_raw),
                             jnp.ceil(phi_raw))
    t0 = jnp.floor(theta_raw).astype(jnp.int32) & (THETA_BINS - 1)
    t1 = jnp.ceil(theta_raw).astype(jnp.int32) & (THETA_BINS - 1)
    p0 = jnp.floor(phi_raw).astype(jnp.int32) & (PHI_BINS - 1)
    p1 = jnp.ceil(phi_raw).astype(jnp.int32) & (PHI_BINS - 1)
    f0 = jnp.abs(tf)
    f1 = jnp.abs(1.0 - tf)
    f2 = jnp.abs(pf)
    f3 = jnp.abs(1.0 - pf)

    it = lax.broadcasted_iota(jnp.int32, (THETA_BINS, D, D), 0)
    T = (jnp.where(it == t0[None], f0[None], 0.0)
         + jnp.where(it == t1[None], f1[None], 0.0)) * mag[None]
    P = (jnp.where(it == p0[None], f2[None], 0.0)
         + jnp.where(it == p1[None], f3[None], 0.0))
    zt = jnp.zeros((THETA_BINS, 8, D), jnp.float32)
    zb = jnp.zeros((THETA_BINS, 10, D), jnp.float32)
    Tp = jnp.concatenate([zt, T, zb], axis=1)
    Pp = jnp.concatenate([zt, P, zb], axis=1)
    hist = (Tp[:, None] * Pp[None, :]).reshape(NB, 96, D)

    ir = lax.broadcasted_iota(jnp.int32, (D, D), 0)
    ic = lax.broadcasted_iota(jnp.int32, (D, D), 1)
    nw = (jnp.abs(ir - ic) <= PAD).astype(jnp.float32)
    yw = jax.lax.dot(hist.reshape(NB * 96, D), nw).reshape(NB, 96, D)
    jr = lax.broadcasted_iota(jnp.int32, (D, 96), 0)
    jc = lax.broadcasted_iota(jnp.int32, (D, 96), 1)
    nh = (jnp.abs(jc - 8 - jr) <= PAD).astype(jnp.float32)
    return jnp.concatenate(
        [jax.lax.dot(nh, yw[b])[None] for b in range(NB)], axis=0)


def _fused_body(xpad_ref, o_ref, ring_ref, s_ref):
    i = pl.program_id(0)

    @pl.when(i == 0)
    def _init():
        ring_ref[...] = jnp.zeros((BLOCK, NB, D, D), jnp.float32)
        s_ref[...] = jnp.zeros((NB, D, D), jnp.float32)

    pln = _hw_plane(xpad_ref, i)
    slot = lax.rem(i, BLOCK)
    old = ring_ref[slot]
    s_new = s_ref[...] + pln - old
    s_ref[...] = s_new
    ring_ref[slot] = pln

    @pl.when(i >= PAD)
    def _emit():
        z = i - PAD
        ih = lax.broadcasted_iota(jnp.int32, (D, D), 0)
        iw = lax.broadcasted_iota(jnp.int32, (D, D), 1)
        inv2 = (1.0 / _cnt(z)) / (_cnt(ih) * _cnt(iw))
        o_ref[:, 0] = s_new * inv2[None]


@functools.partial(jax.jit, static_argnames=("interpret",))
def _hog(x, weight, interpret=False):
    del weight
    xr = x.astype(jnp.bfloat16).astype(jnp.float32)
    xpad = jnp.pad(xr, ((8, 15), (8, 8), (8, 8)))
    out = pl.pallas_call(
        _fused_body,
        grid=(STEPS,),
        in_specs=[pl.BlockSpec((N + 23, N + 16, N + 16),
                               lambda i: (0, 0, 0))],
        out_specs=pl.BlockSpec(
            (NB, 1, D, D),
            lambda i: (0, jnp.where(i < PAD, 0, i - PAD), 0, 0)),
        out_shape=jax.ShapeDtypeStruct((NB, D, D, D), jnp.float32),
        scratch_shapes=[
            pltpu.VMEM((BLOCK, NB, D, D), jnp.float32),
            pltpu.VMEM((NB, D, D), jnp.float32),
        ],
        interpret=interpret,
    )(xpad)
    return out


def kernel(x, weight):
    return _hog(x, weight)
